# Initial kernel scaffold; baseline (speedup 1.0000x reference)
import functools

import jax
import jax.numpy as jnp
from jax import lax
from jax.experimental import pallas as pl
from jax.experimental.pallas import tpu as pltpu
from jax.experimental.pallas import tpu_sc as plsc

N = 10000
E = 320000
D = 128
G = 64
EPS = 1e-5

NC = 2
NS = 16
NW = NC * NS
EPW = E // NW
CH = 80
NCHUNK = EPW // CH
RPT = N // NS

_sc_mesh = plsc.VectorSubcoreMesh(core_axis_name="c", subcore_axis_name="s")


@functools.partial(
    pl.kernel,
    mesh=_sc_mesh,
    out_type=jax.ShapeDtypeStruct((NC, N, D), jnp.float32),
    scratch_types=[
        pltpu.VMEM((NCHUNK, CH), jnp.int32),
        pltpu.VMEM((NCHUNK, CH), jnp.int32),
        pltpu.VMEM((CH, D), jnp.float32),
        pltpu.VMEM((CH, D), jnp.float32),
        pltpu.VMEM_SHARED((N, D), jnp.float32),
        pltpu.SemaphoreType.DMA,
        pltpu.SemaphoreType.DMA,
    ],
)
def _edge_acc(yin_hbm, yout_hbm, src_hbm, dst_hbm, zero_hbm, out_hbm,
              src_v, dst_v, rows_a, rows_b, acc_sh, sem_a, sem_b):
    cid = lax.axis_index("c")
    sid = lax.axis_index("s")
    wid = sid * NC + cid

    row0 = sid * RPT
    pltpu.sync_copy(zero_hbm.at[pl.ds(row0, RPT)], acc_sh.at[pl.ds(row0, RPT)])

    pltpu.sync_copy(src_hbm.at[pl.ds(wid * NCHUNK, NCHUNK)], src_v)
    pltpu.sync_copy(dst_hbm.at[pl.ds(wid * NCHUNK, NCHUNK)], dst_v)
    plsc.subcore_barrier()

    def chunk(i, carry):
        src_i = src_v.at[i]
        dst_i = dst_v.at[i]
        ga = pltpu.async_copy(yin_hbm.at[src_i], rows_a, sem_a)
        gb = pltpu.async_copy(yout_hbm.at[dst_i], rows_b, sem_b)
        ga.wait()
        pltpu.sync_copy(rows_a, acc_sh.at[dst_i], add=True)
        gb.wait()
        pltpu.sync_copy(rows_b, acc_sh.at[src_i], add=True)
        return carry

    lax.fori_loop(0, NCHUNK, chunk, 0)
    plsc.subcore_barrier()

    pltpu.sync_copy(acc_sh.at[pl.ds(row0, RPT)],
                    out_hbm.at[cid, pl.ds(row0, RPT)])


def _mm3_body(x_ref, ws_ref, wi_ref, wo_ref, s_ref, yi_ref, yo_ref):
    xv = x_ref[...]
    s_ref[...] = jnp.dot(xv, ws_ref[...], preferred_element_type=jnp.float32)
    yi_ref[...] = jnp.dot(xv, wi_ref[...], preferred_element_type=jnp.float32)
    yo_ref[...] = jnp.dot(xv, wo_ref[...], preferred_element_type=jnp.float32)


_mm3 = pl.pallas_call(
    _mm3_body,
    out_shape=[jax.ShapeDtypeStruct((N, D), jnp.float32)] * 3,
)


def _norm(s_ref, acc_ref, g_ref, b_ref):
    h = jnp.maximum(s_ref[...] + acc_ref[0] + acc_ref[1], 0.0)
    mu = jnp.mean(h, axis=0, keepdims=True)
    var = jnp.mean((h - mu) * (h - mu), axis=0, keepdims=True)
    return (h - mu) * lax.rsqrt(var + EPS) * g_ref[...] + b_ref[...]


def _norm_mm3_body(s_ref, acc_ref, g_ref, b_ref, ws_ref, wi_ref, wo_ref,
                   s2_ref, yi2_ref, yo2_ref):
    h = _norm(s_ref, acc_ref, g_ref, b_ref)
    s2_ref[...] = jnp.dot(h, ws_ref[...], preferred_element_type=jnp.float32)
    yi2_ref[...] = jnp.dot(h, wi_ref[...], preferred_element_type=jnp.float32)
    yo2_ref[...] = jnp.dot(h, wo_ref[...], preferred_element_type=jnp.float32)


_norm_mm3 = pl.pallas_call(
    _norm_mm3_body,
    out_shape=[jax.ShapeDtypeStruct((N, D), jnp.float32)] * 3,
)


def _pool_body(s_ref, acc_ref, g_ref, b_ref, batch_ref, out_ref):
    h = _norm(s_ref, acc_ref, g_ref, b_ref)
    seg = batch_ref[...]
    gids = lax.broadcasted_iota(jnp.int32, (G, N), 0)
    onehot = (gids == seg).astype(jnp.float32)
    sums = jnp.dot(onehot, h, preferred_element_type=jnp.float32)
    counts = jnp.sum(onehot, axis=1, keepdims=True)
    out_ref[...] = sums / jnp.maximum(counts, 1.0)


_pool = pl.pallas_call(
    _pool_body,
    out_shape=jax.ShapeDtypeStruct((G, D), jnp.float32),
)


def kernel(x, edge_index, batch,
           W_self_0, W_in_0, W_out_0, g_0, b_0,
           W_self_1, W_in_1, W_out_1, g_1, b_1,
           W_self_2, W_in_2, W_out_2, g_2, b_2):
    src = edge_index[0].astype(jnp.int32).reshape(NW * NCHUNK, CH)
    dst = edge_index[1].astype(jnp.int32).reshape(NW * NCHUNK, CH)
    zero = jnp.zeros((N, D), jnp.float32)
    batch2 = batch.astype(jnp.int32).reshape(1, N)

    s, yi, yo = _mm3(x, W_self_0, W_in_0, W_out_0)
    acc = _edge_acc(yi, yo, src, dst, zero)
    s, yi, yo = _norm_mm3(s, acc, g_0, b_0, W_self_1, W_in_1, W_out_1)
    acc = _edge_acc(yi, yo, src, dst, zero)
    s, yi, yo = _norm_mm3(s, acc, g_1, b_1, W_self_2, W_in_2, W_out_2)
    acc = _edge_acc(yi, yo, src, dst, zero)
    return _pool(s, acc, g_2, b_2, batch2)


# speedup vs baseline: 6.6928x; 6.6928x over previous
import functools

import jax
import jax.numpy as jnp
from jax import lax
from jax.experimental import pallas as pl
from jax.experimental.pallas import tpu as pltpu
from jax.experimental.pallas import tpu_sc as plsc

N = 10000
E = 320000
D = 128
G = 64
EPS = 1e-5

NC = 2
NS = 16
NW = NC * NS
EPW = E // NW
CH = 80
NCHUNK = EPW // CH
NSTAGE = 5
CPS = NCHUNK // NSTAGE
N_PAD = 10240
RPT = N_PAD // NS

def _edge_acc_body(yin_hbm, yout_hbm, src_hbm, dst_hbm, zero_hbm, out_hbm,
                   src_v, dst_v, rows_a, rows_b, acc_sh, sem_a, sem_b):
    cid = lax.axis_index("c")
    sid = lax.axis_index("s")
    wid = sid * NC + cid

    row0 = sid * RPT
    pltpu.sync_copy(zero_hbm.at[pl.ds(row0, RPT)], acc_sh.at[pl.ds(row0, RPT)])

    plsc.subcore_barrier()

    def stage(j, carry):
        pltpu.sync_copy(src_hbm.at[wid, j], src_v)
        pltpu.sync_copy(dst_hbm.at[wid, j], dst_v)

        def chunk(i, c):
            src_i = src_v.at[i]
            dst_i = dst_v.at[i]
            ga = pltpu.async_copy(yin_hbm.at[src_i], rows_a, sem_a)
            gb = pltpu.async_copy(yout_hbm.at[dst_i], rows_b, sem_b)
            ga.wait()
            pltpu.sync_copy(rows_a, acc_sh.at[dst_i], add=True)
            gb.wait()
            pltpu.sync_copy(rows_b, acc_sh.at[src_i], add=True)
            return c

        return lax.fori_loop(0, CPS, chunk, carry)

    lax.fori_loop(0, NSTAGE, stage, 0)
    plsc.subcore_barrier()

    pltpu.sync_copy(acc_sh.at[pl.ds(row0, RPT)],
                    out_hbm.at[cid, pl.ds(row0, RPT)])


@functools.cache
def _edge_acc():
    mesh = plsc.VectorSubcoreMesh(core_axis_name="c", subcore_axis_name="s")
    return pl.kernel(
        _edge_acc_body,
        mesh=mesh,
        out_type=jax.ShapeDtypeStruct((NC, N_PAD, D), jnp.float32),
        scratch_types=[
            pltpu.VMEM((CPS, CH), jnp.int32),
            pltpu.VMEM((CPS, CH), jnp.int32),
            pltpu.VMEM((CH, D), jnp.float32),
            pltpu.VMEM((CH, D), jnp.float32),
            pltpu.VMEM_SHARED((N_PAD, D), jnp.float32),
            pltpu.SemaphoreType.DMA,
            pltpu.SemaphoreType.DMA,
        ],
    )


def _mm3_body(x_ref, ws_ref, wi_ref, wo_ref, s_ref, yi_ref, yo_ref):
    xv = x_ref[...]
    s_ref[...] = jnp.dot(xv, ws_ref[...], preferred_element_type=jnp.float32, precision=lax.Precision.HIGHEST)
    yi_ref[...] = jnp.dot(xv, wi_ref[...], preferred_element_type=jnp.float32, precision=lax.Precision.HIGHEST)
    yo_ref[...] = jnp.dot(xv, wo_ref[...], preferred_element_type=jnp.float32, precision=lax.Precision.HIGHEST)


_mm3 = pl.pallas_call(
    _mm3_body,
    out_shape=[jax.ShapeDtypeStruct((N, D), jnp.float32)] * 3,
)


def _norm_body(s_ref, acc_ref, g_ref, b_ref, h_ref):
    acc = acc_ref[...]
    h = jnp.maximum(s_ref[...] + acc[0, :N] + acc[1, :N], 0.0)
    mu = jnp.mean(h, axis=0, keepdims=True)
    var = jnp.mean((h - mu) * (h - mu), axis=0, keepdims=True)
    h_ref[...] = (h - mu) * lax.rsqrt(var + EPS) * g_ref[...] + b_ref[...]


_norm = pl.pallas_call(
    _norm_body,
    out_shape=jax.ShapeDtypeStruct((N, D), jnp.float32),
)


def _pool_body(h_ref, batch_ref, out_ref):
    h = h_ref[...]
    seg = batch_ref[...]
    gids = lax.broadcasted_iota(jnp.int32, (G, N), 0)
    onehot = (gids == seg).astype(jnp.float32)
    sums = jnp.dot(onehot, h, preferred_element_type=jnp.float32, precision=lax.Precision.HIGHEST)
    counts = jnp.sum(onehot, axis=1, keepdims=True)
    out_ref[...] = sums / jnp.maximum(counts, 1.0)


_pool = pl.pallas_call(
    _pool_body,
    out_shape=jax.ShapeDtypeStruct((G, D), jnp.float32),
)


def kernel(x, edge_index, batch,
           W_self_0, W_in_0, W_out_0, g_0, b_0,
           W_self_1, W_in_1, W_out_1, g_1, b_1,
           W_self_2, W_in_2, W_out_2, g_2, b_2):
    src = edge_index[0].astype(jnp.int32).reshape(NW, NSTAGE, CPS, CH)
    dst = edge_index[1].astype(jnp.int32).reshape(NW, NSTAGE, CPS, CH)
    zero = jnp.zeros((N_PAD, D), jnp.float32)
    batch2 = batch.astype(jnp.int32).reshape(1, N)

    h = x
    for (Ws, Wi, Wo, g, b) in (
        (W_self_0, W_in_0, W_out_0, g_0, b_0),
        (W_self_1, W_in_1, W_out_1, g_1, b_1),
        (W_self_2, W_in_2, W_out_2, g_2, b_2),
    ):
        s, yi, yo = _mm3(h, Ws, Wi, Wo)
        acc = _edge_acc()(yi, yo, src, dst, zero)
        h = _norm(s, acc, g, b)
    return _pool(h, batch2)
